# Initial kernel scaffold; baseline (speedup 1.0000x reference)
import jax
import jax.numpy as jnp
from jax.experimental import pallas as pl


def kernel(hidden_states, routing_indices, routing_weights, gate_up_proj, gate_up_proj_bias, down_proj, down_proj_bias):
    raise NotImplementedError("write your pallas kernel here")


# speedup vs baseline: 22.7426x; 22.7426x over previous
import functools

import jax
import jax.numpy as jnp
from jax import lax
from jax.experimental import pallas as pl
from jax.experimental.pallas import tpu as pltpu
from jax.experimental.pallas import tpu_sc as plsc

NUM_EXPERTS = 8
TOP_K = 2
HIDDEN = 2048
INTER = 2048
ALPHA = 1.702
LIMIT = 7.0

BLK = 256
NB = 24
P = NB * BLK
NT = 2048
N_TILES = (2 * INTER) // NT


def _routing_plan(routing_indices, routing_weights):
    e_flat = routing_indices.reshape(-1).astype(jnp.int32)
    w_flat = routing_weights.reshape(-1)
    onehot = (e_flat[:, None] == jnp.arange(NUM_EXPERTS, dtype=jnp.int32)
              ).astype(jnp.int32)
    counts = onehot.sum(axis=0)
    blocks_e = (counts + BLK - 1) // BLK
    cum_blocks = jnp.cumsum(blocks_e)
    block_start = cum_blocks - blocks_e
    total_blocks = cum_blocks[-1]
    rank = (jnp.cumsum(onehot, axis=0) * onehot).sum(axis=1) - 1
    dest = (block_start[e_flat] * BLK + rank).astype(jnp.int32)
    w_sorted = jnp.zeros((P,), jnp.float32).at[dest].set(w_flat)
    bidx = jnp.arange(NB, dtype=jnp.int32)
    safe_b = jnp.minimum(bidx, total_blocks - 1)
    block_expert = jnp.searchsorted(cum_blocks, safe_b, side="right"
                                    ).astype(jnp.int32)
    block_valid = (bidx < total_blocks).astype(jnp.int32)
    return dest, w_sorted, block_expert, block_valid


def _sc_dispatch(x, src_ids, dest, n_pairs, d):
    info = plsc.get_sparse_core_info()
    nw = info.num_cores * info.num_subcores
    per_w = n_pairs // nw
    ch = 16
    n_ch = per_w // ch
    nbuf = 3
    mesh = plsc.VectorSubcoreMesh(core_axis_name="c", subcore_axis_name="s")

    @functools.partial(
        pl.kernel, mesh=mesh,
        out_type=jax.ShapeDtypeStruct((P, d), jnp.float32),
        scratch_types=[
            [pltpu.VMEM((ch,), jnp.int32) for _ in range(nbuf)],
            [pltpu.VMEM((ch,), jnp.int32) for _ in range(nbuf)],
            [pltpu.VMEM((ch, d), jnp.float32) for _ in range(nbuf)],
            [pltpu.SemaphoreType.DMA for _ in range(nbuf)],
            [pltpu.SemaphoreType.DMA for _ in range(nbuf)],
            [pltpu.SemaphoreType.DMA for _ in range(nbuf)],
        ],
    )
    def k(x_hbm, src_hbm, dest_hbm, out_hbm, didx_vs, sidx_vs, bufs,
          isems, gsems, ssems):
        wid = lax.axis_index("s") * info.num_cores + lax.axis_index("c")
        base = wid * per_w

        def start(c, slot):
            off = base + c * ch
            pltpu.async_copy(dest_hbm.at[pl.ds(off, ch)], didx_vs[slot],
                             isems[slot])
            pltpu.sync_copy(src_hbm.at[pl.ds(off, ch)], sidx_vs[slot])
            pltpu.async_copy(x_hbm.at[sidx_vs[slot]], bufs[slot],
                             gsems[slot])

        def finish(c, slot):
            pltpu.make_async_copy(dest_hbm.at[pl.ds(0, ch)], didx_vs[slot],
                                  isems[slot]).wait()
            pltpu.make_async_copy(x_hbm.at[sidx_vs[slot]], bufs[slot],
                                  gsems[slot]).wait()
            pltpu.async_copy(bufs[slot], out_hbm.at[didx_vs[slot]],
                             ssems[slot])

        def drain(slot):
            pltpu.make_async_copy(bufs[slot], out_hbm.at[didx_vs[slot]],
                                  ssems[slot]).wait()

        for c in range(min(nbuf, n_ch)):
            start(c, c % nbuf)
        for c in range(n_ch):
            slot = c % nbuf
            finish(c, slot)
            nxt = c + nbuf
            if nxt < n_ch:
                drain(slot)
                start(nxt, slot)
        for c in range(max(n_ch - nbuf, 0), n_ch):
            drain(c % nbuf)

    return k(x, src_ids, dest)


def _sc_combine(y2, d0, d1, n_tok, d):
    info = plsc.get_sparse_core_info()
    nw = info.num_cores * info.num_subcores
    tok_per_w = n_tok // nw
    ch = 8
    n_ch = tok_per_w // ch
    nsl = d // 16
    nbuf = 3
    mesh = plsc.VectorSubcoreMesh(core_axis_name="c", subcore_axis_name="s")

    @functools.partial(
        pl.kernel, mesh=mesh,
        out_type=jax.ShapeDtypeStruct((n_tok, d), jnp.float32),
        scratch_types=[
            [pltpu.VMEM((ch,), jnp.int32) for _ in range(nbuf)],
            [pltpu.VMEM((ch,), jnp.int32) for _ in range(nbuf)],
            [pltpu.VMEM((ch, d), jnp.float32) for _ in range(nbuf)],
            [pltpu.VMEM((ch, d), jnp.float32) for _ in range(nbuf)],
            [pltpu.SemaphoreType.DMA for _ in range(nbuf)],
            [pltpu.SemaphoreType.DMA for _ in range(nbuf)],
            [pltpu.SemaphoreType.DMA for _ in range(nbuf)],
        ],
    )
    def k(y_hbm, d0_hbm, d1_hbm, out_hbm, i0_vs, i1_vs, a_vs, b_vs, s0s,
          s1s, wsems):
        wid = lax.axis_index("s") * info.num_cores + lax.axis_index("c")
        base = wid * tok_per_w

        def start(c, slot):
            off = base + c * ch
            pltpu.sync_copy(d0_hbm.at[pl.ds(off, ch)], i0_vs[slot])
            pltpu.sync_copy(d1_hbm.at[pl.ds(off, ch)], i1_vs[slot])
            pltpu.async_copy(y_hbm.at[i0_vs[slot]], a_vs[slot], s0s[slot])
            pltpu.async_copy(y_hbm.at[i1_vs[slot]], b_vs[slot], s1s[slot])

        def finish(c, slot):
            off = base + c * ch
            pltpu.make_async_copy(y_hbm.at[i0_vs[slot]], a_vs[slot],
                                  s0s[slot]).wait()
            pltpu.make_async_copy(y_hbm.at[i1_vs[slot]], b_vs[slot],
                                  s1s[slot]).wait()
            for r in range(ch):
                def lane(j, c2):
                    for u in range(8):
                        sl = pl.ds(j * 128 + u * 16, 16)
                        a_vs[slot][r, sl] = (a_vs[slot][r, sl]
                                             + b_vs[slot][r, sl])
                    return c2
                lax.fori_loop(0, nsl // 8, lane, 0)
            pltpu.async_copy(a_vs[slot], out_hbm.at[pl.ds(off, ch)],
                             wsems[slot])

        for c in range(min(nbuf, n_ch)):
            start(c, c % nbuf)
        for c in range(n_ch):
            slot = c % nbuf
            finish(c, slot)
            nxt = c + nbuf
            if nxt < n_ch:
                pltpu.make_async_copy(a_vs[slot],
                                      out_hbm.at[pl.ds(0, ch)],
                                      wsems[slot]).wait()
                start(nxt, slot)
        for c in range(max(n_ch - nbuf, 0), n_ch):
            slot = c % nbuf
            pltpu.make_async_copy(a_vs[slot], out_hbm.at[pl.ds(0, ch)],
                                  wsems[slot]).wait()

    return k(y2, d0, d1)


def _mlp1_body(be_ref, bv_ref, x_ref, w_ref, bias_ref, comb_ref, out_ref):
    b = pl.program_id(1)

    @pl.when(bv_ref[b] == 1)
    def _():
        h = jnp.dot(x_ref[...], w_ref[0], precision=lax.Precision.DEFAULT,
                    preferred_element_type=jnp.float32)
        h = h + bias_ref[0]
        gate = jnp.minimum(h, LIMIT)
        glu = gate * jax.nn.sigmoid(gate * ALPHA)
        upb = jnp.clip(h, -LIMIT, LIMIT) + 1.0
        upb = jnp.concatenate([upb[:, 1:], upb[:, :1]], axis=1)
        t = (glu * upb).astype(jnp.bfloat16)
        out_ref[...] = jnp.dot(t, comb_ref[...],
                               precision=lax.Precision.DEFAULT,
                               preferred_element_type=jnp.float32
                               ).astype(out_ref.dtype)


def _mlp1(x_sorted, gate_up_proj, gate_up_proj_bias, comb, block_expert,
          block_valid):
    grid = (N_TILES, NB)
    return pl.pallas_call(
        _mlp1_body,
        grid_spec=pltpu.PrefetchScalarGridSpec(
            num_scalar_prefetch=2,
            grid=grid,
            in_specs=[
                pl.BlockSpec((BLK, HIDDEN), lambda n, b, be, bv: (b, 0)),
                pl.BlockSpec((1, HIDDEN, NT), lambda n, b, be, bv: (be[b], 0, n)),
                pl.BlockSpec((1, 1, NT), lambda n, b, be, bv: (be[b], 0, n)),
                pl.BlockSpec((NT, NT // 2), lambda n, b, be, bv: (0, 0)),
            ],
            out_specs=pl.BlockSpec((BLK, NT // 2), lambda n, b, be, bv: (b, n)),
        ),
        out_shape=jax.ShapeDtypeStruct((P, INTER), jnp.bfloat16),
    )(block_expert, block_valid, x_sorted, gate_up_proj,
      gate_up_proj_bias[:, None, :], comb)


def _mlp2_body(be_ref, bv_ref, g_ref, w_ref, bias_ref, wt_ref, out_ref):
    b = pl.program_id(0)

    @pl.when(bv_ref[b] == 1)
    def _():
        y = jnp.dot(g_ref[...].astype(jnp.float32), w_ref[0],
                    precision=lax.Precision.DEFAULT,
                    preferred_element_type=jnp.float32)
        y = y + bias_ref[0]
        out_ref[...] = y * wt_ref[...]


def _mlp2(gated, down_proj, down_proj_bias, w_sorted, block_expert, block_valid):
    grid = (NB,)
    return pl.pallas_call(
        _mlp2_body,
        grid_spec=pltpu.PrefetchScalarGridSpec(
            num_scalar_prefetch=2,
            grid=grid,
            in_specs=[
                pl.BlockSpec((BLK, INTER), lambda b, be, bv: (b, 0)),
                pl.BlockSpec((1, INTER, HIDDEN), lambda b, be, bv: (be[b], 0, 0)),
                pl.BlockSpec((1, 1, HIDDEN), lambda b, be, bv: (be[b], 0, 0)),
                pl.BlockSpec((BLK, 1), lambda b, be, bv: (b, 0)),
            ],
            out_specs=pl.BlockSpec((BLK, HIDDEN), lambda b, be, bv: (b, 0)),
        ),
        out_shape=jax.ShapeDtypeStruct((P, HIDDEN), jnp.float32),
    )(block_expert, block_valid, gated, down_proj,
      down_proj_bias[:, None, :], w_sorted)


def kernel(hidden_states, routing_indices, routing_weights, gate_up_proj,
           gate_up_proj_bias, down_proj, down_proj_bias):
    B, S, H = hidden_states.shape
    x = hidden_states.reshape(-1, H)
    n_tok = x.shape[0]

    dest, w_sorted, block_expert, block_valid = _routing_plan(
        routing_indices, routing_weights)

    n_pairs = dest.shape[0]
    src_ids = jnp.arange(n_pairs, dtype=jnp.int32) // TOP_K
    x_sorted = _sc_dispatch(x, src_ids, dest, n_pairs, H)
    comb = (jnp.arange(NT)[:, None] == 2 * jnp.arange(NT // 2)[None, :]
            ).astype(jnp.bfloat16)
    gated = _mlp1(x_sorted, gate_up_proj, gate_up_proj_bias, comb,
                  block_expert, block_valid)
    y2 = _mlp2(gated, down_proj, down_proj_bias, w_sorted[:, None],
               block_expert, block_valid)

    d0 = dest[0::2]
    d1 = dest[1::2]
    out = _sc_combine(y2, d0, d1, n_tok, H)
    return out.reshape(B, S, H)
